# Initial kernel scaffold; baseline (speedup 1.0000x reference)
import jax
import jax.numpy as jnp
from jax.experimental import pallas as pl


def kernel(pred_logits, pred_boxes, orig_target_sizes):
    raise NotImplementedError("write your pallas kernel here")


# speedup vs baseline: 7.5723x; 7.5723x over previous
import jax
import jax.numpy as jnp
from jax.experimental import pallas as pl
from jax.experimental.pallas import tpu as pltpu

_C = 80
_K = 300
_KPAD = 304
_ROWS = 12500
_ROWS_PAD = 12544
_CHUNKS = 98
_M1_ROWS = 104
_NEG = float("-inf")
_QCHUNK = 2048
_QPAD = 20480
_BIG = 2 ** 30


def _topk_kernel(logits_ref, labels_ref, scores_ref, qidx_ref,
                 m1_ref, m1i_ref):
    m1_ref[...] = jnp.full((_M1_ROWS, 128), -2.0, jnp.float32)
    m1i_ref[...] = jnp.zeros((_M1_ROWS, 128), jnp.int32)

    def init_body(j, carry):
        sg = jax.nn.sigmoid(logits_ref[0, pl.ds(j * 128, 128), :])
        logits_ref[0, pl.ds(j * 128, 128), :] = sg
        m1_ref[pl.ds(j, 1), :] = jnp.max(sg, axis=0, keepdims=True)
        m1i_ref[pl.ds(j, 1), :] = jnp.argmax(
            sg, axis=0, keepdims=True).astype(jnp.int32)
        return carry

    jax.lax.fori_loop(0, _CHUNKS, init_body, 0)

    j_iota = jax.lax.broadcasted_iota(jnp.int32, (_M1_ROWS, 128), 0)
    l_iota = jax.lax.broadcasted_iota(jnp.int32, (_M1_ROWS, 128), 1)
    lane_iota_row = jax.lax.broadcasted_iota(jnp.int32, (1, 128), 1)

    def body(i, carry):
        m1 = m1_ref[...]
        vmax = jnp.max(m1)
        fgrid = (j_iota * 128 + m1i_ref[...]) * 128 + l_iota
        fwin = jnp.min(jnp.where(m1 == vmax, fgrid, _BIG))
        r = fwin // 128
        l = fwin - r * 128
        j = r // 128

        row = logits_ref[0, pl.ds(r, 1), :]
        logits_ref[0, pl.ds(r, 1), :] = jnp.where(
            lane_iota_row == l, -1.0, row)
        chunk2 = logits_ref[0, pl.ds(j * 128, 128), :]
        m1_ref[pl.ds(j, 1), :] = jnp.max(chunk2, axis=0, keepdims=True)
        m1i_ref[pl.ds(j, 1), :] = jnp.argmax(
            chunk2, axis=0, keepdims=True).astype(jnp.int32)

        q = fwin // _C
        c = fwin - q * _C
        labels_ref[0, pl.ds(i, 1), :] = jnp.full((1, 1), c, jnp.int32)
        qidx_ref[0, pl.ds(i, 1), :] = jnp.full((1, 1), q, jnp.int32)
        scores_ref[0, pl.ds(i, 1), :] = jnp.full((1, 1), vmax, jnp.float32)
        return carry

    jax.lax.fori_loop(0, _K, body, 0)


def _boxes_kernel(boxes_t_ref, qs_ref, sizes_ref, out_ref):
    qs = qs_ref[0, :, :]
    acc = jnp.zeros((4, _KPAD), jnp.float32)

    for t in range(_QPAD // _QCHUNK):
        base = t * _QCHUNK
        riota = jax.lax.broadcasted_iota(
            jnp.int32, (_QCHUNK, 1), 0) + base
        onehot = (riota == qs).astype(jnp.float32)
        blk = boxes_t_ref[0, :, base:base + _QCHUNK]
        acc = acc + jnp.dot(blk, onehot,
                            preferred_element_type=jnp.float32)

    cx = acc[0:1, :]
    cy = acc[1:2, :]
    w = acc[2:3, :]
    h = acc[3:4, :]
    w_sz = sizes_ref[0, 0, 0]
    h_sz = sizes_ref[0, 0, 1]
    x1 = (cx - 0.5 * w) * w_sz
    y1 = (cy - 0.5 * h) * h_sz
    x2 = (cx + 0.5 * w) * w_sz
    y2 = (cy + 0.5 * h) * h_sz
    out_ref[0, :, :] = jnp.concatenate(
        [x1, y1, x2, y2, jnp.zeros((4, _KPAD), jnp.float32)], axis=0)


def kernel(pred_logits, pred_boxes, orig_target_sizes):
    B, Q, C = pred_logits.shape
    flat = pred_logits.reshape(B, _ROWS, 128)
    flat = jnp.pad(flat, ((0, 0), (0, _ROWS_PAD - _ROWS), (0, 0)),
                   constant_values=_NEG)

    labels, scores, qidx = pl.pallas_call(
        _topk_kernel,
        grid=(B,),
        in_specs=[
            pl.BlockSpec((1, _ROWS_PAD, 128), lambda b: (b, 0, 0)),
        ],
        out_specs=[
            pl.BlockSpec((1, _KPAD, 1), lambda b: (b, 0, 0)),
            pl.BlockSpec((1, _KPAD, 1), lambda b: (b, 0, 0)),
            pl.BlockSpec((1, _KPAD, 1), lambda b: (b, 0, 0)),
        ],
        out_shape=[
            jax.ShapeDtypeStruct((B, _KPAD, 1), jnp.int32),
            jax.ShapeDtypeStruct((B, _KPAD, 1), jnp.float32),
            jax.ShapeDtypeStruct((B, _KPAD, 1), jnp.int32),
        ],
        scratch_shapes=[pltpu.VMEM((_M1_ROWS, 128), jnp.float32),
                        pltpu.VMEM((_M1_ROWS, 128), jnp.int32)],
        compiler_params=pltpu.CompilerParams(
            dimension_semantics=("arbitrary",)),
    )(flat)

    boxes_t = jnp.pad(jnp.transpose(pred_boxes, (0, 2, 1)),
                      ((0, 0), (0, 0), (0, _QPAD - Q)))
    qs_lane = jnp.transpose(qidx, (0, 2, 1))
    sizes3 = orig_target_sizes.reshape(B, 1, 2)

    boxes8 = pl.pallas_call(
        _boxes_kernel,
        grid=(B,),
        in_specs=[
            pl.BlockSpec((1, 4, _QPAD), lambda b: (b, 0, 0)),
            pl.BlockSpec((1, 1, _KPAD), lambda b: (b, 0, 0)),
            pl.BlockSpec((1, 1, 2), lambda b: (b, 0, 0)),
        ],
        out_specs=pl.BlockSpec((1, 8, _KPAD), lambda b: (b, 0, 0)),
        out_shape=jax.ShapeDtypeStruct((B, 8, _KPAD), jnp.float32),
        compiler_params=pltpu.CompilerParams(
            dimension_semantics=("arbitrary",)),
    )(boxes_t, qs_lane, sizes3)

    boxes = jnp.transpose(boxes8[:, :4, :_K], (0, 2, 1))
    return (labels[:, :_K, 0], boxes, scores[:, :_K, 0])


# speedup vs baseline: 7.5728x; 1.0001x over previous
import jax
import jax.numpy as jnp
from jax.experimental import pallas as pl
from jax.experimental.pallas import tpu as pltpu

_C = 80
_K = 300
_KPAD = 304
_ROWS = 12500
_ROWS_PAD = 12544
_CHUNKS = 98
_M1_ROWS = 104
_NEG = float("-inf")
_QCHUNK = 2048
_QPAD = 20480
_BIG = 2 ** 30


def _topk_kernel(logits_ref, labels_ref, scores_ref, qidx_ref,
                 m1_ref, m1i_ref):
    m1_ref[...] = jnp.full((_M1_ROWS, 128), -2.0, jnp.float32)
    m1i_ref[...] = jnp.zeros((_M1_ROWS, 128), jnp.int32)

    def init_body(j, carry):
        sg = jax.nn.sigmoid(logits_ref[0, pl.ds(j * 128, 128), :])
        logits_ref[0, pl.ds(j * 128, 128), :] = sg
        m1_ref[pl.ds(j, 1), :] = jnp.max(sg, axis=0, keepdims=True)
        m1i_ref[pl.ds(j, 1), :] = jnp.argmax(
            sg, axis=0, keepdims=True).astype(jnp.int32)
        return carry

    jax.lax.fori_loop(0, _CHUNKS, init_body, 0)

    j_iota = jax.lax.broadcasted_iota(jnp.int32, (_M1_ROWS, 128), 0)
    l_iota = jax.lax.broadcasted_iota(jnp.int32, (_M1_ROWS, 128), 1)
    lane_iota_row = jax.lax.broadcasted_iota(jnp.int32, (1, 128), 1)

    def body(i, carry):
        m1 = m1_ref[...]
        vmax = jnp.max(m1)
        fgrid = (j_iota * 128 + m1i_ref[...]) * 128 + l_iota
        fwin = jnp.min(jnp.where(m1 == vmax, fgrid, _BIG))
        r = fwin // 128
        l = fwin - r * 128
        j = r // 128

        row = logits_ref[0, pl.ds(r, 1), :]
        logits_ref[0, pl.ds(r, 1), :] = jnp.where(
            lane_iota_row == l, -1.0, row)
        chunk2 = logits_ref[0, pl.ds(j * 128, 128), :]
        m1_ref[pl.ds(j, 1), :] = jnp.max(chunk2, axis=0, keepdims=True)
        m1i_ref[pl.ds(j, 1), :] = jnp.argmax(
            chunk2, axis=0, keepdims=True).astype(jnp.int32)

        q = fwin // _C
        c = fwin - q * _C
        labels_ref[0, pl.ds(i, 1), :] = jnp.full((1, 1), c, jnp.int32)
        qidx_ref[0, pl.ds(i, 1), :] = jnp.full((1, 1), q, jnp.int32)
        scores_ref[0, pl.ds(i, 1), :] = jnp.full((1, 1), vmax, jnp.float32)
        return carry

    jax.lax.fori_loop(0, _K, body, 0)


def _boxes_kernel(boxes_t_ref, qs_ref, sizes_ref, out_ref):
    qs = qs_ref[0, :, :]
    acc = jnp.zeros((4, _KPAD), jnp.float32)

    for t in range(_QPAD // _QCHUNK):
        base = t * _QCHUNK
        riota = jax.lax.broadcasted_iota(
            jnp.int32, (_QCHUNK, 1), 0) + base
        onehot = (riota == qs).astype(jnp.float32)
        blk = boxes_t_ref[0, :, base:base + _QCHUNK]
        acc = acc + jnp.dot(blk, onehot,
                            preferred_element_type=jnp.float32)

    cx = acc[0:1, :]
    cy = acc[1:2, :]
    w = acc[2:3, :]
    h = acc[3:4, :]
    w_sz = sizes_ref[0, 0, 0]
    h_sz = sizes_ref[0, 0, 1]
    x1 = (cx - 0.5 * w) * w_sz
    y1 = (cy - 0.5 * h) * h_sz
    x2 = (cx + 0.5 * w) * w_sz
    y2 = (cy + 0.5 * h) * h_sz
    out_ref[0, :, :] = jnp.concatenate(
        [x1, y1, x2, y2, jnp.zeros((4, _KPAD), jnp.float32)], axis=0)


def kernel(pred_logits, pred_boxes, orig_target_sizes):
    B, Q, C = pred_logits.shape
    flat = pred_logits.reshape(B, _ROWS, 128)
    flat = jnp.pad(flat, ((0, 0), (0, _ROWS_PAD - _ROWS), (0, 0)),
                   constant_values=_NEG)

    labels, scores, qidx = pl.pallas_call(
        _topk_kernel,
        grid=(B,),
        in_specs=[
            pl.BlockSpec((1, _ROWS_PAD, 128), lambda b: (b, 0, 0)),
        ],
        out_specs=[
            pl.BlockSpec((1, _KPAD, 1), lambda b: (b, 0, 0)),
            pl.BlockSpec((1, _KPAD, 1), lambda b: (b, 0, 0)),
            pl.BlockSpec((1, _KPAD, 1), lambda b: (b, 0, 0)),
        ],
        out_shape=[
            jax.ShapeDtypeStruct((B, _KPAD, 1), jnp.int32),
            jax.ShapeDtypeStruct((B, _KPAD, 1), jnp.float32),
            jax.ShapeDtypeStruct((B, _KPAD, 1), jnp.int32),
        ],
        scratch_shapes=[pltpu.VMEM((_M1_ROWS, 128), jnp.float32),
                        pltpu.VMEM((_M1_ROWS, 128), jnp.int32)],
        compiler_params=pltpu.CompilerParams(
            dimension_semantics=("parallel",)),
    )(flat)

    boxes_t = jnp.pad(jnp.transpose(pred_boxes, (0, 2, 1)),
                      ((0, 0), (0, 0), (0, _QPAD - Q)))
    qs_lane = jnp.transpose(qidx, (0, 2, 1))
    sizes3 = orig_target_sizes.reshape(B, 1, 2)

    boxes8 = pl.pallas_call(
        _boxes_kernel,
        grid=(B,),
        in_specs=[
            pl.BlockSpec((1, 4, _QPAD), lambda b: (b, 0, 0)),
            pl.BlockSpec((1, 1, _KPAD), lambda b: (b, 0, 0)),
            pl.BlockSpec((1, 1, 2), lambda b: (b, 0, 0)),
        ],
        out_specs=pl.BlockSpec((1, 8, _KPAD), lambda b: (b, 0, 0)),
        out_shape=jax.ShapeDtypeStruct((B, 8, _KPAD), jnp.float32),
        compiler_params=pltpu.CompilerParams(
            dimension_semantics=("parallel",)),
    )(boxes_t, qs_lane, sizes3)

    boxes = jnp.transpose(boxes8[:, :4, :_K], (0, 2, 1))
    return (labels[:, :_K, 0], boxes, scores[:, :_K, 0])
